# baseline (speedup 1.0000x reference)
import functools

import jax
import jax.numpy as jnp
from jax import lax
from jax.experimental import pallas as pl
from jax.experimental.pallas import tpu as pltpu
from jax.experimental.pallas import tpu_sc as plsc

HIDDEN = 4096
N_TOK = 16384
SC_TOK = 12288
TC_TOK = N_TOK - SC_TOK
NUM_CORES = 2
NUM_SUBCORES = 16
NW = NUM_CORES * NUM_SUBCORES
CHUNK = 8
NBUF = 2
TC_RPS = 16


def _build(n_tok):
    b_per_w = n_tok // NW
    n_chunks = b_per_w // CHUNK
    mesh = plsc.VectorSubcoreMesh(core_axis_name="c", subcore_axis_name="s")

    @functools.partial(
        pl.kernel,
        mesh=mesh,
        out_type=jax.ShapeDtypeStruct((n_tok, HIDDEN), jnp.float32),
        scratch_types=[
            pltpu.VMEM((n_chunks, CHUNK), jnp.int32),
            pltpu.VMEM((NBUF, CHUNK, HIDDEN), jnp.float32),
        ] + [pltpu.SemaphoreType.DMA] * (2 * NBUF),
    )
    def emb(ids_hbm, table_hbm, out_hbm, idx_v, rows_v, *sems):
        gsem = list(sems[:NBUF])
        osem = list(sems[NBUF:])
        wid = lax.axis_index("s") * NUM_CORES + lax.axis_index("c")
        base = wid * b_per_w
        pltpu.sync_copy(ids_hbm.at[pl.ds(wid * n_chunks, n_chunks)], idx_v)

        def g_desc(j, b):
            return pltpu.make_async_copy(
                table_hbm.at[idx_v.at[j]],
                rows_v.at[b],
                gsem[b],
            )

        def o_desc(j, b):
            return pltpu.make_async_copy(
                rows_v.at[b],
                out_hbm.at[pl.ds(base + j * CHUNK, CHUNK)],
                osem[b],
            )

        for b in range(NBUF):
            g_desc(b, b).start()

        def outer(i, carry):
            j0 = i * NBUF
            for b in range(NBUF):
                j = j0 + b
                g_desc(j, b).wait()
                o_desc(j, b).start()

                @pl.when(j + NBUF < n_chunks)
                def _():
                    o_desc(j, b).wait()
                    g_desc(j + NBUF, b).start()

            return carry

        lax.fori_loop(0, n_chunks // NBUF, outer, 0)

        for b in range(NBUF):
            o_desc(n_chunks - NBUF + b, b).wait()

    return emb


_emb = _build(SC_TOK)


def _tc_body(ids_ref, *refs):
    out = refs[TC_RPS]
    for k in range(TC_RPS):
        out[k, :] = refs[k][0, 0, :]


def _mk_map(k):
    def f(i, ids):
        return (ids[i * TC_RPS + k], 0, 0)
    return f


def _tc_gather(ids_tail, table):
    n = ids_tail.shape[0]
    return pl.pallas_call(
        _tc_body,
        grid_spec=pltpu.PrefetchScalarGridSpec(
            num_scalar_prefetch=1,
            grid=(n // TC_RPS,),
            in_specs=[pl.BlockSpec((1, 1, HIDDEN), _mk_map(k)) for k in range(TC_RPS)],
            out_specs=pl.BlockSpec((TC_RPS, HIDDEN), lambda i, ids: (i, 0)),
        ),
        out_shape=jax.ShapeDtypeStruct((n, HIDDEN), jnp.float32),
    )(ids_tail, *([table.reshape(table.shape[0], 1, HIDDEN)] * TC_RPS))


def kernel(input_ids, embed_tokens_weight):
    b, s = input_ids.shape
    ids_flat = input_ids.reshape(-1).astype(jnp.int32)
    sc_ids = ids_flat[:SC_TOK].reshape(SC_TOK // CHUNK, CHUNK)
    sc_out = _emb(sc_ids, embed_tokens_weight)
    tc_out = _tc_gather(ids_flat[SC_TOK:], embed_tokens_weight)
    out = jnp.concatenate([sc_out, tc_out], axis=0)
    return out.reshape(b, s, HIDDEN)
